# baseline (speedup 1.0000x reference)
import functools

import jax
import jax.numpy as jnp
from jax import lax
from jax.experimental import pallas as pl
from jax.experimental.pallas import tpu as pltpu
from jax.experimental.pallas import tpu_sc as plsc

N_NODES = 10000
D_FEAT = 128
HID = 16

NC = 2
NS = 16
NW = NC * NS
CHUNK = 128
CHUNKS_PER_TILE = 80
E_PAD = NW * CHUNKS_PER_TILE * CHUNK
NBUF = 8
NGRP = CHUNKS_PER_TILE // NBUF
ACC_ROWS = 10240
ACC_PER_TILE = ACC_ROWS // NS
DUMP_ROW = 10000

_mesh = plsc.VectorSubcoreMesh(core_axis_name="c", subcore_axis_name="s")


@functools.partial(
    pl.kernel,
    out_type=jax.ShapeDtypeStruct((NC, ACC_ROWS, HID), jnp.float32),
    mesh=_mesh,
    scratch_types=[
        pltpu.VMEM((CHUNKS_PER_TILE, CHUNK), jnp.int32),
        pltpu.VMEM((CHUNKS_PER_TILE, CHUNK), jnp.int32),
        [pltpu.VMEM((CHUNK, HID), jnp.float32) for _ in range(NBUF)],
        pltpu.VMEM((ACC_PER_TILE, HID), jnp.float32),
        pltpu.VMEM_SHARED((ACC_ROWS, HID), jnp.float32),
        [pltpu.SemaphoreType.DMA for _ in range(NBUF)],
    ],
    compiler_params=pltpu.CompilerParams(use_tc_tiling_on_sc=False),
)
def _sc_scatter_rows(h_hbm, src_hbm, dst_hbm, zero_hbm, out_hbm,
                     src_v, dst_v, rows_v, stage_v, acc_sh, sems):
    c = lax.axis_index("c")
    s = lax.axis_index("s")
    wid = c * NS + s

    sl = pl.ds(s * ACC_PER_TILE, ACC_PER_TILE)
    pltpu.sync_copy(zero_hbm.at[sl], stage_v)
    pltpu.sync_copy(stage_v, acc_sh.at[sl])
    pltpu.sync_copy(src_hbm.at[wid], src_v)
    pltpu.sync_copy(dst_hbm.at[wid], dst_v)
    plsc.subcore_barrier()

    for b in range(NBUF):
        pltpu.async_copy(h_hbm.at[src_v.at[b]], rows_v[b], sems[b])

    def grp(g, carry):
        for b in range(NBUF):
            j = g * NBUF + b
            pltpu.make_async_copy(h_hbm.at[src_v.at[j]], rows_v[b],
                                  sems[b]).wait()
            pltpu.sync_copy(rows_v[b], acc_sh.at[dst_v.at[j]], add=True)

            @pl.when(j + NBUF < CHUNKS_PER_TILE)
            def _():
                pltpu.async_copy(h_hbm.at[src_v.at[j + NBUF]], rows_v[b],
                                 sems[b])
        return carry
    lax.fori_loop(0, NGRP, grp, 0)
    plsc.subcore_barrier()

    pltpu.sync_copy(acc_sh.at[sl], stage_v)
    pltpu.sync_copy(stage_v, out_hbm.at[c, sl])


@functools.partial(
    pl.kernel,
    out_type=jax.ShapeDtypeStruct((NC, ACC_ROWS, HID), jnp.float32),
    mesh=_mesh,
    scratch_types=[
        pltpu.VMEM((CHUNKS_PER_TILE, CHUNK), jnp.int32),
        pltpu.VMEM((CHUNK, HID), jnp.float32),
        pltpu.VMEM((ACC_PER_TILE, HID), jnp.float32),
        pltpu.VMEM_SHARED((ACC_ROWS, HID), jnp.float32),
        [pltpu.SemaphoreType.DMA for _ in range(NBUF)],
    ],
    compiler_params=pltpu.CompilerParams(use_tc_tiling_on_sc=False),
)
def _sc_degree(dst_hbm, ones_hbm, zero_hbm, out_hbm, dst_v, ones_v, stage_v,
               acc_sh, sems):
    c = lax.axis_index("c")
    s = lax.axis_index("s")
    wid = c * NS + s

    sl = pl.ds(s * ACC_PER_TILE, ACC_PER_TILE)
    pltpu.sync_copy(zero_hbm.at[sl], stage_v)
    pltpu.sync_copy(stage_v, acc_sh.at[sl])
    pltpu.sync_copy(ones_hbm, ones_v)
    pltpu.sync_copy(dst_hbm.at[wid], dst_v)
    plsc.subcore_barrier()

    for b in range(NBUF):
        pltpu.async_copy(ones_v, acc_sh.at[dst_v.at[b]], sems[b], add=True)

    def grp(g, carry):
        for b in range(NBUF):
            j = g * NBUF + b
            pltpu.make_async_copy(ones_v, acc_sh.at[dst_v.at[j - NBUF]],
                                  sems[b]).wait()
            pltpu.async_copy(ones_v, acc_sh.at[dst_v.at[j]], sems[b], add=True)
        return carry
    lax.fori_loop(1, NGRP, grp, 0)
    for b in range(NBUF):
        j = (NGRP - 1) * NBUF + b
        pltpu.make_async_copy(ones_v, acc_sh.at[dst_v.at[j]], sems[b]).wait()
    plsc.subcore_barrier()

    pltpu.sync_copy(acc_sh.at[sl], stage_v)
    pltpu.sync_copy(stage_v, out_hbm.at[c, sl])


BLK = 1000


def _dinv(deg_ref):
    return lax.rsqrt(deg_ref[0] + deg_ref[1] + 1.0)


def _tc_h1(x_ref, w_ref, deg_ref, o_ref):
    h = jnp.dot(x_ref[...], w_ref[...], preferred_element_type=jnp.float32)
    o_ref[...] = h * _dinv(deg_ref)


def _tc_mid(s_ref, hp_ref, deg_ref, w_ref, b_ref, o_ref):
    dinv = _dinv(deg_ref)
    z = dinv * (s_ref[0] + s_ref[1] + hp_ref[...]) + b_ref[...]
    z = jnp.maximum(z, 0.0)
    o_ref[...] = jnp.dot(z, w_ref[...], preferred_element_type=jnp.float32) * dinv


def _tc_out(s_ref, hp_ref, deg_ref, b_ref, o_ref):
    o_ref[...] = _dinv(deg_ref) * (s_ref[0] + s_ref[1] + hp_ref[...]) + b_ref[...]


_part_spec = pl.BlockSpec((NC, BLK, HID), lambda i: (0, i, 0))
_row_spec = pl.BlockSpec((BLK, HID), lambda i: (i, 0))

_h1_call = pl.pallas_call(
    _tc_h1,
    grid=(N_NODES // BLK,),
    in_specs=[
        pl.BlockSpec((BLK, D_FEAT), lambda i: (i, 0)),
        pl.BlockSpec((D_FEAT, HID), lambda i: (0, 0)),
        _part_spec,
    ],
    out_specs=_row_spec,
    out_shape=jax.ShapeDtypeStruct((N_NODES, HID), jnp.float32),
)

_mid_call = pl.pallas_call(
    _tc_mid,
    grid=(N_NODES // BLK,),
    in_specs=[
        _part_spec,
        _row_spec,
        _part_spec,
        pl.BlockSpec((HID, HID), lambda i: (0, 0)),
        pl.BlockSpec((1, HID), lambda i: (0, 0)),
    ],
    out_specs=_row_spec,
    out_shape=jax.ShapeDtypeStruct((N_NODES, HID), jnp.float32),
)

_out_call = pl.pallas_call(
    _tc_out,
    grid=(N_NODES // BLK,),
    in_specs=[
        _part_spec,
        _row_spec,
        _part_spec,
        pl.BlockSpec((1, HID), lambda i: (0, 0)),
    ],
    out_specs=_row_spec,
    out_shape=jax.ShapeDtypeStruct((N_NODES, HID), jnp.float32),
)


def kernel(x, edge_index, W1, b1, W2, b2):
    src = edge_index[0].astype(jnp.int32)
    dst = edge_index[1].astype(jnp.int32)
    pad = E_PAD - src.shape[0]
    src_p = jnp.pad(src, (0, pad)).reshape(NW, CHUNKS_PER_TILE, CHUNK)
    dst_p = jnp.pad(dst, (0, pad), constant_values=DUMP_ROW)
    dst_p = dst_p.reshape(NW, CHUNKS_PER_TILE, CHUNK)

    zeros_hbm = jnp.zeros((ACC_ROWS, HID), jnp.float32)
    ones_hbm = jnp.ones((CHUNK, HID), jnp.float32)

    deg = _sc_degree(dst_p, ones_hbm, zeros_hbm)
    h1p = _h1_call(x, W1, deg)
    s1 = _sc_scatter_rows(h1p, src_p, dst_p, zeros_hbm)
    h2p = _mid_call(s1, h1p, deg, W2, b1.reshape(1, HID))
    s2 = _sc_scatter_rows(h2p, src_p, dst_p, zeros_hbm)
    return _out_call(s2, h2p, deg, b2.reshape(1, HID))


# speedup vs baseline: 1.4329x; 1.4329x over previous
import functools

import jax
import jax.numpy as jnp
from jax import lax
from jax.experimental import pallas as pl
from jax.experimental.pallas import tpu as pltpu
from jax.experimental.pallas import tpu_sc as plsc

N_NODES = 10000
D_FEAT = 128
HID = 16

NC = 2
NS = 16
NW = NC * NS
CHUNK = 128
CHUNKS_PER_TILE = 80
E_PAD = NW * CHUNKS_PER_TILE * CHUNK
NBUF = 8
NGRP = CHUNKS_PER_TILE // NBUF
ACC_ROWS = 10240
ACC_PER_TILE = ACC_ROWS // NS
DUMP_ROW = 10000

_mesh = plsc.VectorSubcoreMesh(core_axis_name="c", subcore_axis_name="s")


@functools.partial(
    pl.kernel,
    out_type=jax.ShapeDtypeStruct((NC, ACC_ROWS, HID), jnp.float32),
    mesh=_mesh,
    scratch_types=[
        pltpu.VMEM((CHUNKS_PER_TILE, CHUNK), jnp.int32),
        pltpu.VMEM((CHUNKS_PER_TILE, CHUNK), jnp.int32),
        [pltpu.VMEM((CHUNK, HID), jnp.float32) for _ in range(NBUF)],
        pltpu.VMEM((ACC_PER_TILE, HID), jnp.float32),
        pltpu.VMEM_SHARED((ACC_ROWS, HID), jnp.float32),
        pltpu.VMEM_SHARED((N_NODES, HID), jnp.float32),
        [pltpu.SemaphoreType.DMA for _ in range(NBUF)],
    ],
    compiler_params=pltpu.CompilerParams(use_tc_tiling_on_sc=False),
)
def _sc_scatter_rows(h_hbm, src_hbm, dst_hbm, zero_hbm, out_hbm,
                     src_v, dst_v, rows_v, stage_v, acc_sh, table_sh, sems):
    c = lax.axis_index("c")
    s = lax.axis_index("s")
    wid = c * NS + s

    sl = pl.ds(s * ACC_PER_TILE, ACC_PER_TILE)
    pltpu.sync_copy(zero_hbm.at[sl], stage_v)
    pltpu.sync_copy(stage_v, acc_sh.at[sl])
    pltpu.sync_copy(src_hbm.at[wid], src_v)
    pltpu.sync_copy(dst_hbm.at[wid], dst_v)
    tsl = pl.ds(s * (N_NODES // NS), N_NODES // NS)
    pltpu.sync_copy(h_hbm.at[tsl], table_sh.at[tsl])
    plsc.subcore_barrier()

    for b in range(NBUF):
        pltpu.async_copy(table_sh.at[src_v.at[b]], rows_v[b], sems[b])

    def grp(g, carry):
        for b in range(NBUF):
            j = g * NBUF + b
            pltpu.make_async_copy(table_sh.at[src_v.at[j]], rows_v[b],
                                  sems[b]).wait()
            pltpu.sync_copy(rows_v[b], acc_sh.at[dst_v.at[j]], add=True)

            @pl.when(j + NBUF < CHUNKS_PER_TILE)
            def _():
                pltpu.async_copy(table_sh.at[src_v.at[j + NBUF]], rows_v[b],
                                 sems[b])
        return carry
    lax.fori_loop(0, NGRP, grp, 0)
    plsc.subcore_barrier()

    pltpu.sync_copy(acc_sh.at[sl], stage_v)
    pltpu.sync_copy(stage_v, out_hbm.at[c, sl])


@functools.partial(
    pl.kernel,
    out_type=jax.ShapeDtypeStruct((NC, ACC_ROWS, HID), jnp.float32),
    mesh=_mesh,
    scratch_types=[
        pltpu.VMEM((CHUNKS_PER_TILE, CHUNK), jnp.int32),
        pltpu.VMEM((CHUNK, HID), jnp.float32),
        pltpu.VMEM((ACC_PER_TILE, HID), jnp.float32),
        pltpu.VMEM_SHARED((ACC_ROWS, HID), jnp.float32),
        [pltpu.SemaphoreType.DMA for _ in range(NBUF)],
    ],
    compiler_params=pltpu.CompilerParams(use_tc_tiling_on_sc=False),
)
def _sc_degree(dst_hbm, ones_hbm, zero_hbm, out_hbm, dst_v, ones_v, stage_v,
               acc_sh, sems):
    c = lax.axis_index("c")
    s = lax.axis_index("s")
    wid = c * NS + s

    sl = pl.ds(s * ACC_PER_TILE, ACC_PER_TILE)
    pltpu.sync_copy(zero_hbm.at[sl], stage_v)
    pltpu.sync_copy(stage_v, acc_sh.at[sl])
    pltpu.sync_copy(ones_hbm, ones_v)
    pltpu.sync_copy(dst_hbm.at[wid], dst_v)
    plsc.subcore_barrier()

    for b in range(NBUF):
        pltpu.async_copy(ones_v, acc_sh.at[dst_v.at[b]], sems[b], add=True)

    def grp(g, carry):
        for b in range(NBUF):
            j = g * NBUF + b
            pltpu.make_async_copy(ones_v, acc_sh.at[dst_v.at[j - NBUF]],
                                  sems[b]).wait()
            pltpu.async_copy(ones_v, acc_sh.at[dst_v.at[j]], sems[b], add=True)
        return carry
    lax.fori_loop(1, NGRP, grp, 0)
    for b in range(NBUF):
        j = (NGRP - 1) * NBUF + b
        pltpu.make_async_copy(ones_v, acc_sh.at[dst_v.at[j]], sems[b]).wait()
    plsc.subcore_barrier()

    pltpu.sync_copy(acc_sh.at[sl], stage_v)
    pltpu.sync_copy(stage_v, out_hbm.at[c, sl])


BLK = 1000


def _dinv(deg_ref):
    return lax.rsqrt(deg_ref[0] + deg_ref[1] + 1.0)


def _tc_h1(x_ref, w_ref, deg_ref, o_ref):
    h = jnp.dot(x_ref[...], w_ref[...], preferred_element_type=jnp.float32)
    o_ref[...] = h * _dinv(deg_ref)


def _tc_mid(s_ref, hp_ref, deg_ref, w_ref, b_ref, o_ref):
    dinv = _dinv(deg_ref)
    z = dinv * (s_ref[0] + s_ref[1] + hp_ref[...]) + b_ref[...]
    z = jnp.maximum(z, 0.0)
    o_ref[...] = jnp.dot(z, w_ref[...], preferred_element_type=jnp.float32) * dinv


def _tc_out(s_ref, hp_ref, deg_ref, b_ref, o_ref):
    o_ref[...] = _dinv(deg_ref) * (s_ref[0] + s_ref[1] + hp_ref[...]) + b_ref[...]


_part_spec = pl.BlockSpec((NC, BLK, HID), lambda i: (0, i, 0))
_row_spec = pl.BlockSpec((BLK, HID), lambda i: (i, 0))

_h1_call = pl.pallas_call(
    _tc_h1,
    grid=(N_NODES // BLK,),
    in_specs=[
        pl.BlockSpec((BLK, D_FEAT), lambda i: (i, 0)),
        pl.BlockSpec((D_FEAT, HID), lambda i: (0, 0)),
        _part_spec,
    ],
    out_specs=_row_spec,
    out_shape=jax.ShapeDtypeStruct((N_NODES, HID), jnp.float32),
)

_mid_call = pl.pallas_call(
    _tc_mid,
    grid=(N_NODES // BLK,),
    in_specs=[
        _part_spec,
        _row_spec,
        _part_spec,
        pl.BlockSpec((HID, HID), lambda i: (0, 0)),
        pl.BlockSpec((1, HID), lambda i: (0, 0)),
    ],
    out_specs=_row_spec,
    out_shape=jax.ShapeDtypeStruct((N_NODES, HID), jnp.float32),
)

_out_call = pl.pallas_call(
    _tc_out,
    grid=(N_NODES // BLK,),
    in_specs=[
        _part_spec,
        _row_spec,
        _part_spec,
        pl.BlockSpec((1, HID), lambda i: (0, 0)),
    ],
    out_specs=_row_spec,
    out_shape=jax.ShapeDtypeStruct((N_NODES, HID), jnp.float32),
)


def kernel(x, edge_index, W1, b1, W2, b2):
    src = edge_index[0].astype(jnp.int32)
    dst = edge_index[1].astype(jnp.int32)
    pad = E_PAD - src.shape[0]
    src_p = jnp.pad(src, (0, pad)).reshape(NW, CHUNKS_PER_TILE, CHUNK)
    dst_p = jnp.pad(dst, (0, pad), constant_values=DUMP_ROW)
    dst_p = dst_p.reshape(NW, CHUNKS_PER_TILE, CHUNK)

    zeros_hbm = jnp.zeros((ACC_ROWS, HID), jnp.float32)
    ones_hbm = jnp.ones((CHUNK, HID), jnp.float32)

    deg = _sc_degree(dst_p, ones_hbm, zeros_hbm)
    h1p = _h1_call(x, W1, deg)
    s1 = _sc_scatter_rows(h1p, src_p, dst_p, zeros_hbm)
    h2p = _mid_call(s1, h1p, deg, W2, b1.reshape(1, HID))
    s2 = _sc_scatter_rows(h2p, src_p, dst_p, zeros_hbm)
    return _out_call(s2, h2p, deg, b2.reshape(1, HID))


# speedup vs baseline: 2.3826x; 1.6628x over previous
import functools

import jax
import jax.numpy as jnp
from jax import lax
from jax.experimental import pallas as pl
from jax.experimental.pallas import tpu as pltpu
from jax.experimental.pallas import tpu_sc as plsc

N_NODES = 10000
D_FEAT = 128
HID = 16

NC = 2
NS = 16
NW = NC * NS
CHUNK = 128
CHUNKS_PER_TILE = 80
E_PAD = NW * CHUNKS_PER_TILE * CHUNK
NBUF = 8
NGRP = CHUNKS_PER_TILE // NBUF
RING = 8
HALF = RING // 2
ACC_ROWS = 10240
ACC_PER_TILE = ACC_ROWS // NS
DUMP_ROW = 10000

_mesh = plsc.VectorSubcoreMesh(core_axis_name="c", subcore_axis_name="s")


@functools.partial(
    pl.kernel,
    out_type=jax.ShapeDtypeStruct((NC, ACC_ROWS, HID), jnp.float32),
    mesh=_mesh,
    scratch_types=[
        pltpu.VMEM((CHUNKS_PER_TILE, CHUNK), jnp.int32),
        pltpu.VMEM((CHUNKS_PER_TILE, CHUNK), jnp.int32),
        [pltpu.VMEM((CHUNK, HID), jnp.float32) for _ in range(RING)],
        pltpu.VMEM((ACC_PER_TILE, HID), jnp.float32),
        pltpu.VMEM_SHARED((ACC_ROWS, HID), jnp.float32),
        pltpu.VMEM_SHARED((ACC_ROWS, HID), jnp.float32),
        [pltpu.SemaphoreType.DMA for _ in range(RING)],
        [pltpu.SemaphoreType.DMA for _ in range(RING)],
    ],
    compiler_params=pltpu.CompilerParams(use_tc_tiling_on_sc=False),
)
def _sc_scatter_rows(h_hbm, ei_hbm, zero_hbm, out_hbm,
                     src_v, dst_v, rows_v, stage_v, acc_sh, table_sh,
                     gsems, ssems):
    c = lax.axis_index("c")
    s = lax.axis_index("s")
    wid = c * NS + s

    sl = pl.ds(s * ACC_PER_TILE, ACC_PER_TILE)
    tsl = pl.ds(s * (N_NODES // NS), N_NODES // NS)
    d0 = pltpu.async_copy(zero_hbm.at[sl], stage_v, gsems[0])
    d1 = pltpu.async_copy(ei_hbm.at[0, wid], src_v, gsems[1])
    d2 = pltpu.async_copy(ei_hbm.at[1, wid], dst_v, gsems[2])
    d3 = pltpu.async_copy(h_hbm.at[tsl], table_sh.at[tsl], gsems[3])
    d0.wait()
    pltpu.sync_copy(stage_v, acc_sh.at[sl])
    d1.wait()
    d2.wait()
    d3.wait()
    plsc.subcore_barrier()

    for b in range(HALF):
        pltpu.async_copy(table_sh.at[src_v.at[b]], rows_v[b], gsems[b])

    def grp(g, carry):
        for k in range(RING):
            j = g * RING + k
            pltpu.make_async_copy(table_sh.at[src_v.at[j]], rows_v[k],
                                  gsems[k]).wait()
            pltpu.async_copy(rows_v[k], acc_sh.at[dst_v.at[j]], ssems[k],
                             add=True)
            kc = (k + HALF) % RING

            @pl.when(j >= HALF)
            def _():
                pltpu.make_async_copy(rows_v[kc],
                                      acc_sh.at[dst_v.at[j - HALF]],
                                      ssems[kc]).wait()

            @pl.when(j + HALF < CHUNKS_PER_TILE)
            def _():
                pltpu.async_copy(table_sh.at[src_v.at[j + HALF]], rows_v[kc],
                                 gsems[kc])
        return carry
    lax.fori_loop(0, CHUNKS_PER_TILE // RING, grp, 0)
    for i in range(HALF):
        j = CHUNKS_PER_TILE - HALF + i
        b = j % RING
        pltpu.make_async_copy(rows_v[b], acc_sh.at[dst_v.at[j]],
                              ssems[b]).wait()
    plsc.subcore_barrier()

    pltpu.sync_copy(acc_sh.at[sl], out_hbm.at[c, sl])


@functools.partial(
    pl.kernel,
    out_type=jax.ShapeDtypeStruct((NC, ACC_ROWS, HID), jnp.float32),
    mesh=_mesh,
    scratch_types=[
        pltpu.VMEM((CHUNKS_PER_TILE, CHUNK), jnp.int32),
        pltpu.VMEM((CHUNK, HID), jnp.float32),
        pltpu.VMEM((ACC_PER_TILE, HID), jnp.float32),
        pltpu.VMEM_SHARED((ACC_ROWS, HID), jnp.float32),
        [pltpu.SemaphoreType.DMA for _ in range(NBUF)],
    ],
    compiler_params=pltpu.CompilerParams(use_tc_tiling_on_sc=False),
)
def _sc_degree(ei_hbm, ones_hbm, zero_hbm, out_hbm, dst_v, ones_v, stage_v,
               acc_sh, sems):
    c = lax.axis_index("c")
    s = lax.axis_index("s")
    wid = c * NS + s

    sl = pl.ds(s * ACC_PER_TILE, ACC_PER_TILE)
    d0 = pltpu.async_copy(zero_hbm.at[sl], stage_v, sems[0])
    d1 = pltpu.async_copy(ones_hbm, ones_v, sems[1])
    d2 = pltpu.async_copy(ei_hbm.at[1, wid], dst_v, sems[2])
    d0.wait()
    pltpu.sync_copy(stage_v, acc_sh.at[sl])
    d1.wait()
    d2.wait()
    plsc.subcore_barrier()

    for b in range(NBUF):
        pltpu.async_copy(ones_v, acc_sh.at[dst_v.at[b]], sems[b], add=True)

    def grp(g, carry):
        for b in range(NBUF):
            j = g * NBUF + b
            pltpu.make_async_copy(ones_v, acc_sh.at[dst_v.at[j - NBUF]],
                                  sems[b]).wait()
            pltpu.async_copy(ones_v, acc_sh.at[dst_v.at[j]], sems[b], add=True)
        return carry
    lax.fori_loop(1, NGRP, grp, 0)
    for b in range(NBUF):
        j = (NGRP - 1) * NBUF + b
        pltpu.make_async_copy(ones_v, acc_sh.at[dst_v.at[j]], sems[b]).wait()
    plsc.subcore_barrier()

    pltpu.sync_copy(acc_sh.at[sl], out_hbm.at[c, sl])


PACK = 128 // HID
PR_N = N_NODES * HID // 128
PR_ACC = ACC_ROWS * HID // 128


def _dinv_packed(deg_ref):
    d = deg_ref[0, :PR_N] + deg_ref[1, :PR_N] + 1.0
    return lax.rsqrt(d)


def _tc_h1(xr_ref, w_ref, deg_ref, o_ref):
    h = jnp.dot(xr_ref[...], w_ref[...], preferred_element_type=jnp.float32)
    o_ref[...] = h * _dinv_packed(deg_ref)


def _tc_mid(s_ref, hp_ref, deg_ref, w_ref, b_ref, o_ref):
    dinv = _dinv_packed(deg_ref)
    z = dinv * (s_ref[0, :PR_N] + s_ref[1, :PR_N] + hp_ref[...]) + b_ref[...]
    z = jnp.maximum(z, 0.0)
    o_ref[...] = jnp.dot(z, w_ref[...], preferred_element_type=jnp.float32) * dinv


def _tc_out(s_ref, hp_ref, deg_ref, b_ref, o_ref):
    o_ref[...] = (_dinv_packed(deg_ref)
                  * (s_ref[0, :PR_N] + s_ref[1, :PR_N] + hp_ref[...])
                  + b_ref[...])


_h1_call = pl.pallas_call(
    _tc_h1,
    out_shape=jax.ShapeDtypeStruct((PR_N, 128), jnp.float32),
)

_mid_call = pl.pallas_call(
    _tc_mid,
    out_shape=jax.ShapeDtypeStruct((PR_N, 128), jnp.float32),
)

_out_call = pl.pallas_call(
    _tc_out,
    out_shape=jax.ShapeDtypeStruct((PR_N, 128), jnp.float32),
)


def kernel(x, edge_index, W1, b1, W2, b2):
    ei = edge_index.astype(jnp.int32)
    pad = E_PAD - ei.shape[1]
    ei_p = jnp.pad(ei, ((0, 0), (0, pad)), constant_values=DUMP_ROW)
    ei_p = ei_p.reshape(2, NW, CHUNKS_PER_TILE, CHUNK)

    zeros_hbm = jnp.zeros((ACC_ROWS, HID), jnp.float32)
    ones_hbm = jnp.ones((CHUNK, HID), jnp.float32)
    xr = x.reshape(PR_N, PACK * D_FEAT)
    w1big = jnp.kron(jnp.eye(PACK, dtype=jnp.float32), W1)
    w2big = jnp.kron(jnp.eye(PACK, dtype=jnp.float32), W2)
    b1t = jnp.tile(b1, PACK).reshape(1, 128)
    b2t = jnp.tile(b2, PACK).reshape(1, 128)

    deg = _sc_degree(ei_p, ones_hbm, zeros_hbm)
    deg_p = deg.reshape(NC, PR_ACC, 128)
    h1p = _h1_call(xr, w1big, deg_p)
    s1 = _sc_scatter_rows(h1p.reshape(N_NODES, HID), ei_p, zeros_hbm)
    h2p = _mid_call(s1.reshape(NC, PR_ACC, 128), h1p, deg_p, w2big, b1t)
    s2 = _sc_scatter_rows(h2p.reshape(N_NODES, HID), ei_p, zeros_hbm)
    out = _out_call(s2.reshape(NC, PR_ACC, 128), h2p, deg_p, b2t)
    return out.reshape(N_NODES, HID)
